# baseline (speedup 1.0000x reference)
import functools
import math

import jax
import jax.numpy as jnp
from jax import lax
from jax.experimental import pallas as pl
from jax.experimental.pallas import tpu as pltpu
from jax.experimental.pallas import tpu_sc as plsc

VB = 1000000
D_MODEL = 64
_SCALE = 1.0 / math.sqrt(D_MODEL)
NC = 2
NS = 16
NW = NC * NS
LANES = 16
CH = 128
NBUF = 4
NB1 = 2
SW = 256
NSLAB = VB // SW
TAILW = VB - NSLAB * SW
SPW1 = NSLAB // NW + 1


@functools.lru_cache(maxsize=None)
def _build_detile():
    mesh = plsc.VectorSubcoreMesh(core_axis_name="c", subcore_axis_name="s")

    @functools.partial(
        pl.kernel,
        mesh=mesh,
        compiler_params=pltpu.CompilerParams(needs_layout_passes=False),
        out_type=jax.ShapeDtypeStruct((VB, 2 * D_MODEL), jnp.float32),
        scratch_types=[
            pltpu.VMEM((D_MODEL, SW + 1), jnp.float32) for _ in range(NB1)
        ]
        + [pltpu.VMEM((SW, 2 * D_MODEL), jnp.float32) for _ in range(NB1)]
        + [pltpu.SemaphoreType.DMA for _ in range(2 * NB1)],
    )
    def detile(tt_hbm, tail_hbm, t_hbm, *rest):
        sbufs = rest[:NB1]
        obufs = rest[NB1:2 * NB1]
        gsems = rest[2 * NB1:3 * NB1]
        ssems = rest[3 * NB1:]
        wid = lax.axis_index("s") * NC + lax.axis_index("c")
        iota = lax.iota(jnp.int32, LANES)
        rowvecs = [iota + c4 * LANES for c4 in range(D_MODEL // LANES)]

        def read_slab(slab, b):
            for dh in range(8):
                pltpu.async_copy(
                    tt_hbm.at[pl.ds(dh * 8, 8), pl.ds(slab * SW, SW)],
                    sbufs[b].at[pl.ds(dh * 8, 8), pl.ds(0, SW)],
                    gsems[b],
                )

        def wait_read(b):
            for dh in range(8):
                pltpu.make_async_copy(
                    tt_hbm.at[pl.ds(0, 8), pl.ds(0, SW)],
                    sbufs[b].at[pl.ds(dh * 8, 8), pl.ds(0, SW)],
                    gsems[b],
                ).wait()

        def transpose(b):
            @plsc.parallel_loop(0, SW, unroll=8)
            def _(vl):
                col = jnp.full((LANES,), vl, dtype=jnp.int32)
                for c4 in range(D_MODEL // LANES):
                    vec = plsc.load_gather(sbufs[b], [rowvecs[c4], col])
                    vec = vec * _SCALE
                    obufs[b][vl, pl.ds(c4 * LANES, LANES)] = vec
                    obufs[b][vl, pl.ds(D_MODEL + c4 * LANES, LANES)] = vec

        def write_slab(slab, b):
            pltpu.async_copy(
                obufs[b],
                t_hbm.at[pl.ds(slab * SW, SW), :],
                ssems[b],
            )

        def wait_write(b):
            pltpu.make_async_copy(
                obufs[b],
                t_hbm.at[pl.ds(0, SW), :],
                ssems[b],
            ).wait()

        def prologue(k, b):
            slab = k * NW + wid

            @pl.when(slab < NSLAB)
            def _():
                read_slab(slab, b)

        for b in range(NB1):
            prologue(b, b)

        @pl.loop(0, SPW1 // NB1 + 1)
        def _(kk):
            for b in range(NB1):
                k = kk * NB1 + b
                slab = k * NW + wid

                @pl.when(slab < NSLAB)
                def _():
                    wait_read(b)

                    @pl.when(k >= NB1)
                    def _():
                        wait_write(b)

                    transpose(b)
                    nslab = (k + NB1) * NW + wid

                    @pl.when(nslab < NSLAB)
                    def _():
                        read_slab(nslab, b)

                    write_slab(slab, b)

        for b in range(NB1):
            wait_write(b)

        @pl.when(wid == 0)
        def _():
            pltpu.sync_copy(tail_hbm, obufs[0].at[pl.ds(0, TAILW), :])
            pltpu.sync_copy(
                obufs[0].at[pl.ds(0, TAILW), :],
                t_hbm.at[pl.ds(NSLAB * SW, TAILW), :],
            )

    return detile


@functools.lru_cache(maxsize=None)
def _build_gather(nch, nbh):
    nblk = (NW * nch // nbh) * 8 * nbh
    mesh = plsc.VectorSubcoreMesh(core_axis_name="c", subcore_axis_name="s")

    @functools.partial(
        pl.kernel,
        mesh=mesh,
        compiler_params=pltpu.CompilerParams(
            use_tc_tiling_on_sc=False, needs_layout_passes=False
        ),
        out_type=jax.ShapeDtypeStruct((nblk, 8, CH), jnp.float32),
        scratch_types=[
            pltpu.VMEM((nch, CH), jnp.int32),
        ]
        + [pltpu.VMEM((CH, 2 * D_MODEL), jnp.float32) for _ in range(NBUF)]
        + [pltpu.VMEM((D_MODEL, CH + 1), jnp.float32) for _ in range(NBUF)]
        + [pltpu.SemaphoreType.DMA for _ in range(2 * NBUF)],
    )
    def emb(idx_hbm, t_hbm, out_hbm, idx_v, *rest):
        gbufs = rest[:NBUF]
        obufs = rest[NBUF:2 * NBUF]
        gsems = rest[2 * NBUF:3 * NBUF]
        ssems = rest[3 * NBUF:]
        wid = lax.axis_index("s") * NC + lax.axis_index("c")
        pltpu.sync_copy(idx_hbm.at[wid], idx_v)

        iota = lax.iota(jnp.int32, LANES)
        rowvecs = [iota + c4 * LANES for c4 in range(D_MODEL // LANES)]

        def gather(jn, b):
            pltpu.async_copy(t_hbm.at[idx_v.at[jn]], gbufs[b], gsems[b])

        def wait_gather(b):
            pltpu.make_async_copy(
                t_hbm.at[idx_v.at[0]], gbufs[b], gsems[b]
            ).wait()

        def store(j, b):
            cidx = wid * nch + j
            l = cidx // nbh
            bh = lax.rem(cidx, nbh)
            blk0 = (l * 8 + 0) * nbh + bh
            for dh in range(8):
                pltpu.async_copy(
                    obufs[b].at[pl.ds(dh * 8, 8), pl.ds(0, CH)],
                    out_hbm.at[blk0 + dh * nbh],
                    ssems[b],
                )

        def wait_store(b):
            for dh in range(8):
                pltpu.make_async_copy(
                    obufs[b].at[pl.ds(dh * 8, 8), pl.ds(0, CH)],
                    out_hbm.at[0],
                    ssems[b],
                ).wait()

        def refill(b, jn):
            @pl.when(jn < nch)
            def _():
                wait_store(b)
                gather(jn, b)

        def transpose(b):
            g = gbufs[b]
            o = obufs[b]

            @plsc.parallel_loop(0, CH, unroll=8)
            def _(bb):
                col = jnp.full((LANES,), bb, dtype=jnp.int32)
                for c4 in range(D_MODEL // LANES):
                    vec = g[bb, pl.ds(c4 * LANES, LANES)]
                    plsc.store_scatter(o, [rowvecs[c4], col], vec)

        for b in range(NBUF):
            gather(b, b)

        @pl.loop(0, nch // NBUF)
        def _(k):
            j0 = k * NBUF
            for b in range(NBUF):
                wait_gather(b)
                transpose(b)
                store(j0 + b, b)
                if b >= 1:
                    refill(b - 1, j0 + NBUF + b - 1)
            refill(NBUF - 1, j0 + 2 * NBUF - 1)

        for b in range(NBUF):
            wait_store(b)

    return emb


def kernel(x, table):
    b, l = x.shape
    bt = b * l
    nch = bt // (NW * CH)
    nbh = b // CH
    xt = x.T.reshape(NW, nch, CH)
    tt = table[NSLAB * SW:] * _SCALE
    tail = jnp.concatenate([tt, tt], axis=1)
    t = _build_detile()(table.T, tail)
    out = _build_gather(nch, nbh)(xt, t)
    out = out.reshape(l, 8, nbh, 8, CH)
    out = out.transpose(2, 4, 0, 1, 3).reshape(b, l, D_MODEL)
    return out


# speedup vs baseline: 2.0956x; 2.0956x over previous
import functools
import math

import jax
import jax.numpy as jnp
from jax import lax
from jax.experimental import pallas as pl
from jax.experimental.pallas import tpu as pltpu
from jax.experimental.pallas import tpu_sc as plsc

VB = 1000000
D_MODEL = 64
_SCALE = 1.0 / math.sqrt(D_MODEL)
NC = 2
NS = 16
NW = NC * NS
LANES = 16
CH = 128
NBUF = 4
NB1 = 4
SW = 128
NSLAB = VB // SW
TAILW = VB - NSLAB * SW
SPW1 = NSLAB // NW + 1


@functools.lru_cache(maxsize=None)
def _build_detile():
    mesh = plsc.VectorSubcoreMesh(core_axis_name="c", subcore_axis_name="s")

    @functools.partial(
        pl.kernel,
        mesh=mesh,
        compiler_params=pltpu.CompilerParams(needs_layout_passes=False),
        out_type=jax.ShapeDtypeStruct((VB, 2 * D_MODEL), jnp.float32),
        scratch_types=[
            pltpu.VMEM((D_MODEL, SW), jnp.float32) for _ in range(NB1)
        ]
        + [pltpu.VMEM((SW, 2 * D_MODEL), jnp.float32) for _ in range(NB1)]
        + [pltpu.SemaphoreType.DMA for _ in range(2 * NB1)],
    )
    def detile(tt_hbm, tail_hbm, t_hbm, *rest):
        sbufs = rest[:NB1]
        obufs = rest[NB1:2 * NB1]
        gsems = rest[2 * NB1:3 * NB1]
        ssems = rest[3 * NB1:]
        wid = lax.axis_index("s") * NC + lax.axis_index("c")
        iota = lax.iota(jnp.int32, LANES)
        rowvecs = [iota + c4 * LANES for c4 in range(D_MODEL // LANES)]

        def read_slab(slab, b):
            for dh in range(8):
                pltpu.async_copy(
                    tt_hbm.at[pl.ds(dh * 8, 8), pl.ds(slab * SW, SW)],
                    sbufs[b].at[pl.ds(dh * 8, 8), :],
                    gsems[b],
                )

        def wait_read(b):
            for dh in range(8):
                pltpu.make_async_copy(
                    tt_hbm.at[pl.ds(0, 8), pl.ds(0, SW)],
                    sbufs[b].at[pl.ds(dh * 8, 8), :],
                    gsems[b],
                ).wait()

        hivecs = [rv + D_MODEL for rv in rowvecs]

        def transpose(b):
            @plsc.parallel_loop(0, SW, unroll=8)
            def _(vl0):
                vlvec = (vl0 + iota) & (SW - 1)
                for c4 in range(D_MODEL // LANES):
                    vec = plsc.load_gather(sbufs[b], [rowvecs[c4], vlvec])
                    vec = vec * _SCALE
                    plsc.store_scatter(obufs[b], [vlvec, rowvecs[c4]], vec)
                    plsc.store_scatter(obufs[b], [vlvec, hivecs[c4]], vec)

        def write_slab(slab, b):
            pltpu.async_copy(
                obufs[b],
                t_hbm.at[pl.ds(slab * SW, SW), :],
                ssems[b],
            )

        def wait_write(b):
            pltpu.make_async_copy(
                obufs[b],
                t_hbm.at[pl.ds(0, SW), :],
                ssems[b],
            ).wait()

        def prologue(k, b):
            slab = k * NW + wid

            @pl.when(slab < NSLAB)
            def _():
                read_slab(slab, b)

        for b in range(NB1):
            prologue(b, b)

        @pl.loop(0, SPW1 // NB1 + 1)
        def _(kk):
            for b in range(NB1):
                k = kk * NB1 + b
                slab = k * NW + wid

                @pl.when(slab < NSLAB)
                def _():
                    wait_read(b)

                    @pl.when(k >= NB1)
                    def _():
                        wait_write(b)

                    transpose(b)
                    nslab = (k + NB1) * NW + wid

                    @pl.when(nslab < NSLAB)
                    def _():
                        read_slab(nslab, b)

                    write_slab(slab, b)

        for b in range(NB1):
            wait_write(b)

        @pl.when(wid == 0)
        def _():
            pltpu.sync_copy(tail_hbm, obufs[0].at[pl.ds(0, TAILW), :])
            pltpu.sync_copy(
                obufs[0].at[pl.ds(0, TAILW), :],
                t_hbm.at[pl.ds(NSLAB * SW, TAILW), :],
            )

    return detile


@functools.lru_cache(maxsize=None)
def _build_gather(nch, nbh):
    nblk = (NW * nch // nbh) * 8 * nbh
    mesh = plsc.VectorSubcoreMesh(core_axis_name="c", subcore_axis_name="s")

    @functools.partial(
        pl.kernel,
        mesh=mesh,
        compiler_params=pltpu.CompilerParams(
            use_tc_tiling_on_sc=False, needs_layout_passes=False
        ),
        out_type=jax.ShapeDtypeStruct((nblk, 8, CH), jnp.float32),
        scratch_types=[
            pltpu.VMEM((nch, CH), jnp.int32),
        ]
        + [pltpu.VMEM((CH, 2 * D_MODEL), jnp.float32) for _ in range(NBUF)]
        + [pltpu.VMEM((D_MODEL, CH + 1), jnp.float32) for _ in range(NBUF)]
        + [pltpu.SemaphoreType.DMA for _ in range(2 * NBUF)],
    )
    def emb(idx_hbm, t_hbm, out_hbm, idx_v, *rest):
        gbufs = rest[:NBUF]
        obufs = rest[NBUF:2 * NBUF]
        gsems = rest[2 * NBUF:3 * NBUF]
        ssems = rest[3 * NBUF:]
        wid = lax.axis_index("s") * NC + lax.axis_index("c")
        pltpu.sync_copy(idx_hbm.at[wid], idx_v)

        iota = lax.iota(jnp.int32, LANES)
        rowvecs = [iota + c4 * LANES for c4 in range(D_MODEL // LANES)]

        def gather(jn, b):
            pltpu.async_copy(t_hbm.at[idx_v.at[jn]], gbufs[b], gsems[b])

        def wait_gather(b):
            pltpu.make_async_copy(
                t_hbm.at[idx_v.at[0]], gbufs[b], gsems[b]
            ).wait()

        def store(j, b):
            cidx = wid * nch + j
            l = cidx // nbh
            bh = lax.rem(cidx, nbh)
            blk0 = (l * 8 + 0) * nbh + bh
            for dh in range(8):
                pltpu.async_copy(
                    obufs[b].at[pl.ds(dh * 8, 8), pl.ds(0, CH)],
                    out_hbm.at[blk0 + dh * nbh],
                    ssems[b],
                )

        def wait_store(b):
            for dh in range(8):
                pltpu.make_async_copy(
                    obufs[b].at[pl.ds(dh * 8, 8), pl.ds(0, CH)],
                    out_hbm.at[0],
                    ssems[b],
                ).wait()

        def refill(b, jn):
            @pl.when(jn < nch)
            def _():
                wait_store(b)
                gather(jn, b)

        def transpose(b):
            g = gbufs[b]
            o = obufs[b]

            @plsc.parallel_loop(0, CH, unroll=8)
            def _(bb):
                col = jnp.full((LANES,), bb, dtype=jnp.int32)
                for c4 in range(D_MODEL // LANES):
                    vec = g[bb, pl.ds(c4 * LANES, LANES)]
                    plsc.store_scatter(o, [rowvecs[c4], col], vec)

        for b in range(NBUF):
            gather(b, b)

        @pl.loop(0, nch // NBUF)
        def _(k):
            j0 = k * NBUF
            for b in range(NBUF):
                wait_gather(b)
                transpose(b)
                store(j0 + b, b)
                if b >= 1:
                    refill(b - 1, j0 + NBUF + b - 1)
            refill(NBUF - 1, j0 + 2 * NBUF - 1)

        for b in range(NBUF):
            wait_store(b)

    return emb


def kernel(x, table):
    b, l = x.shape
    bt = b * l
    nch = bt // (NW * CH)
    nbh = b // CH
    xt = x.T.reshape(NW, nch, CH)
    tt = table[NSLAB * SW:] * _SCALE
    tail = jnp.concatenate([tt, tt], axis=1)
    t = _build_detile()(table.T, tail)
    out = _build_gather(nch, nbh)(xt, t)
    out = out.reshape(l, 8, nbh, 8, CH)
    out = out.transpose(2, 4, 0, 1, 3).reshape(b, l, D_MODEL)
    return out
